# baseline (speedup 1.0000x reference)
import jax
import jax.numpy as jnp
from jax.experimental import pallas as pl


def _body(ids_ref, w_ref, out_ref):
    mask = ids_ref[...] > 0
    for d in range(out_ref.shape[0]):
        out_ref[d] = jnp.where(mask, w_ref[1, d], w_ref[0, d])


def kernel(id1, W):
    N, J = id1.shape
    D = W.shape[1]
    ids_t = id1.T
    Bi = 4096
    out_t = pl.pallas_call(
        _body,
        grid=(N // Bi,),
        in_specs=[
            pl.BlockSpec((J, Bi), lambda i: (0, i)),
            pl.BlockSpec((2, D), lambda i: (0, 0)),
        ],
        out_specs=pl.BlockSpec((D, J, Bi), lambda i: (0, 0, i)),
        out_shape=jax.ShapeDtypeStruct((D, J, N), jnp.float32),
    )(ids_t, W)
    return out_t.transpose(2, 1, 0)
